# baseline (speedup 1.0000x reference)
import functools

import jax
import jax.numpy as jnp
from jax import lax
from jax.experimental import pallas as pl
from jax.experimental.pallas import tpu as pltpu
from jax.experimental.pallas import tpu_sc as plsc

BATCH = 16384
EMBED_DIM = 32
NUM_CORES = 2
NUM_SUBCORES = 16
NUM_WORKERS = NUM_CORES * NUM_SUBCORES
B_PER_W = BATCH // NUM_WORKERS
CHUNK = 128
N_CHUNKS = B_PER_W // CHUNK

_MESH = plsc.VectorSubcoreMesh(core_axis_name="c", subcore_axis_name="s")


@functools.partial(
    pl.kernel,
    mesh=_MESH,
    out_type=jax.ShapeDtypeStruct((BATCH, EMBED_DIM), jnp.float32),
    scratch_types=[
        pltpu.VMEM((N_CHUNKS, CHUNK), jnp.int32),
        pltpu.VMEM((B_PER_W, EMBED_DIM), jnp.float32),
        pltpu.SemaphoreType.DMA,
    ],
    compiler_params=pltpu.CompilerParams(use_tc_tiling_on_sc=False),
)
def _sc_gather(idx_hbm, table_hbm, out_hbm, idx_v, rows_v, sem):
    wid = lax.axis_index("s") * NUM_CORES + lax.axis_index("c")
    base = wid * B_PER_W
    pltpu.sync_copy(idx_hbm.at[wid], idx_v)
    copies = []
    for j in range(N_CHUNKS):
        copies.append(
            pltpu.async_copy(
                table_hbm.at[idx_v.at[j]],
                rows_v.at[pl.ds(j * CHUNK, CHUNK)],
                sem,
            )
        )
    for c in copies:
        c.wait()
    pltpu.sync_copy(rows_v, out_hbm.at[pl.ds(base, B_PER_W)])


def kernel(topic_labels, embedding_weight):
    idx = topic_labels.astype(jnp.int32).reshape(NUM_WORKERS, N_CHUNKS, CHUNK)
    return _sc_gather(idx, embedding_weight)


# speedup vs baseline: 1.5567x; 1.5567x over previous
import functools

import jax
import jax.numpy as jnp
from jax import lax
from jax.experimental import pallas as pl
from jax.experimental.pallas import tpu as pltpu
from jax.experimental.pallas import tpu_sc as plsc

BATCH = 16384
EMBED_DIM = 32
NUM_CORES = 2
NUM_SUBCORES = 16
NUM_WORKERS = NUM_CORES * NUM_SUBCORES
B_PER_W = BATCH // NUM_WORKERS
GROUP = 16
N_GROUPS = B_PER_W // GROUP

_MESH = plsc.VectorSubcoreMesh(core_axis_name="c", subcore_axis_name="s")


@functools.partial(
    pl.kernel,
    mesh=_MESH,
    out_type=jax.ShapeDtypeStruct((BATCH, EMBED_DIM), jnp.float32),
    scratch_types=[
        pltpu.VMEM((B_PER_W,), jnp.int32),
        pltpu.VMEM((B_PER_W, EMBED_DIM), jnp.float32),
        pltpu.SemaphoreType.DMA,
    ],
    compiler_params=pltpu.CompilerParams(needs_layout_passes=False),
)
def _sc_gather(idx_hbm, table_hbm, out_hbm, idx_v, rows_v, sem):
    wid = lax.axis_index("s") * NUM_CORES + lax.axis_index("c")
    base = wid * B_PER_W
    pltpu.sync_copy(idx_hbm.at[wid], idx_v)

    @pl.loop(0, N_GROUPS)
    def _grp(g):
        iv = idx_v[pl.ds(g * GROUP, GROUP)]
        copies = []
        for t in range(GROUP):
            r = iv[t]
            copies.append(
                pltpu.async_copy(
                    table_hbm.at[r], rows_v.at[g * GROUP + t], sem
                )
            )
        for c in copies:
            c.wait()

    pltpu.sync_copy(rows_v, out_hbm.at[pl.ds(base, B_PER_W)])


def kernel(topic_labels, embedding_weight):
    idx = topic_labels.astype(jnp.int32).reshape(NUM_WORKERS, B_PER_W)
    return _sc_gather(idx, embedding_weight)


# speedup vs baseline: 1.6272x; 1.0453x over previous
import functools

import jax
import jax.numpy as jnp
from jax import lax
from jax.experimental import pallas as pl
from jax.experimental.pallas import tpu as pltpu
from jax.experimental.pallas import tpu_sc as plsc

BATCH = 16384
EMBED_DIM = 32
NUM_CORES = 2
NUM_SUBCORES = 16
NUM_WORKERS = NUM_CORES * NUM_SUBCORES
B_PER_W = BATCH // NUM_WORKERS
GROUP = 16
N_GROUPS = B_PER_W // GROUP

_MESH = plsc.VectorSubcoreMesh(core_axis_name="c", subcore_axis_name="s")


@functools.partial(
    pl.kernel,
    mesh=_MESH,
    out_type=jax.ShapeDtypeStruct((BATCH, EMBED_DIM), jnp.float32),
    scratch_types=[
        pltpu.VMEM((B_PER_W,), jnp.int32),
        pltpu.VMEM((B_PER_W, EMBED_DIM), jnp.float32),
        pltpu.SemaphoreType.DMA,
        pltpu.SemaphoreType.DMA,
        pltpu.SemaphoreType.DMA,
        pltpu.SemaphoreType.DMA,
    ],
    compiler_params=pltpu.CompilerParams(needs_layout_passes=False),
)
def _sc_gather(idx_hbm, table_hbm, out_hbm, idx_v, rows_v, s0, s1, s2, s3):
    wid = lax.axis_index("s") * NUM_CORES + lax.axis_index("c")
    base = wid * B_PER_W
    sems = (s0, s1, s2, s3)
    pltpu.sync_copy(idx_hbm.at[wid], idx_v)

    @pl.loop(0, B_PER_W // 64)
    def _grp(g):
        copies = []
        for q in range(4):
            iv = idx_v[pl.ds(g * 64 + q * GROUP, GROUP)]
            for t in range(GROUP):
                copies.append(
                    pltpu.async_copy(
                        table_hbm.at[iv[t]],
                        rows_v.at[g * 64 + q * GROUP + t],
                        sems[q],
                    )
                )
        for c in copies:
            c.wait()

    pltpu.sync_copy(rows_v, out_hbm.at[pl.ds(base, B_PER_W)])


def kernel(topic_labels, embedding_weight):
    idx = topic_labels.astype(jnp.int32).reshape(NUM_WORKERS, B_PER_W)
    return _sc_gather(idx, embedding_weight)
